# Initial kernel scaffold; baseline (speedup 1.0000x reference)
import jax
import jax.numpy as jnp
from jax.experimental import pallas as pl


def kernel(x, train_features, train_labels):
    raise NotImplementedError("write your pallas kernel here")


# speedup vs baseline: 2.0652x; 2.0652x over previous
import functools

import jax
import jax.numpy as jnp
from jax import lax
from jax.experimental import pallas as pl
from jax.experimental.pallas import tpu as pltpu

Q = 1024
D = 32
K = 5
C = 10
BN = 2048
INF = float("inf")
IMAX = 2**31 - 1


def _body(n_total, n_blocks, x_ref, f_ref, lab_ref, out_ref, run_d, run_p):
    g = pl.program_id(0)

    @pl.when(g == 0)
    def _init():
        run_d[:] = jnp.full((Q, 16), INF, jnp.float32)
        run_p[:] = jnp.full((Q, 16), IMAX, jnp.int32)

    xx = x_ref[:]
    xn = jnp.sum(xx * xx, axis=1, keepdims=True)
    f = f_ref[:]
    yn = jnp.sum(f * f, axis=1).reshape(1, BN)
    prod = lax.dot_general(xx, f, (((1,), (1,)), ((), ())),
                           preferred_element_type=jnp.float32)
    d2 = jnp.maximum(xn + yn - 2.0 * prod, 0.0)

    col = lax.broadcasted_iota(jnp.int32, (1, BN), 1)
    gcol = g * BN + col
    d2 = jnp.where(gcol < n_total, d2, INF)

    lab = lab_ref[0]
    pk = gcol * 16 + lab

    bw_d, bw_p = [], []
    d = d2
    for _ in range(K):
        m = jnp.min(d, axis=1, keepdims=True)
        sel = jnp.min(jnp.where(d == m, jnp.broadcast_to(pk, d.shape), IMAX),
                      axis=1, keepdims=True)
        bw_d.append(m)
        bw_p.append(sel)
        d = jnp.where(pk == sel, INF, d)

    inf1 = jnp.full((Q, 1), INF, jnp.float32)
    imax1 = jnp.full((Q, 1), IMAX, jnp.int32)
    cd = jnp.concatenate([run_d[:, :8]] + bw_d + [inf1] * 3, axis=1)
    cp = jnp.concatenate([run_p[:, :8]] + bw_p + [imax1] * 3, axis=1)
    n_d, n_p = [], []
    for _ in range(K):
        m = jnp.min(cd, axis=1, keepdims=True)
        sel = jnp.min(jnp.where(cd == m, cp, IMAX), axis=1, keepdims=True)
        n_d.append(m)
        n_p.append(sel)
        cd = jnp.where(cp == sel, INF, cd)
    run_d[:] = jnp.concatenate(n_d + [inf1] * 11, axis=1)
    run_p[:] = jnp.concatenate(n_p + [imax1] * 11, axis=1)

    @pl.when(g == n_blocks - 1)
    def _finish():
        dist = [jnp.sqrt(v) for v in n_d]
        s0 = -dist[0]
        e = [jnp.exp(-v - s0) for v in dist]
        tot = e[0] + e[1] + e[2] + e[3] + e[4]
        iota_c = lax.broadcasted_iota(jnp.int32, (Q, C), 1)
        o = jnp.zeros((Q, C), jnp.float32)
        for j in range(K):
            labj = n_p[j] & 15
            o = o + (e[j] / tot) * (labj == iota_c).astype(jnp.float32)
        out_ref[:] = o


def kernel(x, train_features, train_labels):
    n = train_features.shape[0]
    g = -(-n // BN)
    npad = g * BN
    f = jnp.pad(train_features, ((0, npad - n), (0, 0)))
    labs = jnp.pad(train_labels, (0, npad - n)).reshape(g, 1, BN)

    body = functools.partial(_body, n, g)
    return pl.pallas_call(
        body,
        grid=(g,),
        in_specs=[
            pl.BlockSpec((Q, D), lambda i: (0, 0)),
            pl.BlockSpec((BN, D), lambda i: (i, 0)),
            pl.BlockSpec((1, 1, BN), lambda i: (i, 0, 0)),
        ],
        out_specs=pl.BlockSpec((Q, C), lambda i: (0, 0)),
        out_shape=jax.ShapeDtypeStruct((Q, C), jnp.float32),
        scratch_shapes=[
            pltpu.VMEM((Q, 16), jnp.float32),
            pltpu.VMEM((Q, 16), jnp.int32),
        ],
        compiler_params=pltpu.CompilerParams(
            dimension_semantics=("arbitrary",),
        ),
    )(x, f, labs)


# speedup vs baseline: 2.6085x; 1.2630x over previous
import functools

import jax
import jax.numpy as jnp
from jax import lax
from jax.experimental import pallas as pl
from jax.experimental.pallas import tpu as pltpu

Q = 1024
D = 32
K = 5
C = 10
BN = 2048
INF = float("inf")


def _body(n_total, n_blocks, x_ref, f_ref, lab_ref, out_ref, run_d, run_p):
    g = pl.program_id(0)

    @pl.when(g == 0)
    def _init():
        run_d[:] = jnp.full((Q, 16), INF, jnp.float32)
        run_p[:] = jnp.full((Q, 16), INF, jnp.float32)

    xx = x_ref[:]
    xn = jnp.sum(xx * xx, axis=1, keepdims=True)
    f = f_ref[:]
    yn = jnp.sum(f * f, axis=1).reshape(1, BN)
    col = lax.broadcasted_iota(jnp.int32, (1, BN), 1)
    gcol = g * BN + col
    yn = jnp.where(gcol < n_total, yn, INF)
    prod = lax.dot_general(xx, f, (((1,), (1,)), ((), ())),
                           preferred_element_type=jnp.float32)
    d2 = jnp.maximum(xn + yn - 2.0 * prod, 0.0)

    lab = lab_ref[0]
    pk = (gcol * 16 + lab).astype(jnp.float32)

    bw_d, bw_p = [], []
    d = d2
    for j in range(K):
        m = jnp.min(d, axis=1, keepdims=True)
        sel = jnp.min(jnp.where(d == m, pk, INF),
                      axis=1, keepdims=True)
        bw_d.append(m)
        bw_p.append(sel)
        if j < K - 1:
            d = jnp.where(pk == sel, INF, d)

    inf1 = jnp.full((Q, 1), INF, jnp.float32)
    cd = jnp.concatenate([run_d[:, :8]] + bw_d + [inf1] * 3, axis=1)
    cp = jnp.concatenate([run_p[:, :8]] + bw_p + [inf1] * 3, axis=1)
    n_d, n_p = [], []
    for j in range(K):
        m = jnp.min(cd, axis=1, keepdims=True)
        sel = jnp.min(jnp.where(cd == m, cp, INF), axis=1, keepdims=True)
        n_d.append(m)
        n_p.append(sel)
        if j < K - 1:
            cd = jnp.where(cp == sel, INF, cd)
    run_d[:] = jnp.concatenate(n_d + [inf1] * 11, axis=1)
    run_p[:] = jnp.concatenate(n_p + [inf1] * 11, axis=1)

    @pl.when(g == n_blocks - 1)
    def _finish():
        dist = [jnp.sqrt(v) for v in n_d]
        s0 = -dist[0]
        e = [jnp.exp(-v - s0) for v in dist]
        tot = e[0] + e[1] + e[2] + e[3] + e[4]
        iota_c = lax.broadcasted_iota(jnp.int32, (Q, C), 1)
        o = jnp.zeros((Q, C), jnp.float32)
        for j in range(K):
            labj = n_p[j].astype(jnp.int32) & 15
            o = o + (e[j] / tot) * (labj == iota_c).astype(jnp.float32)
        out_ref[:] = o


def kernel(x, train_features, train_labels):
    n = train_features.shape[0]
    g = -(-n // BN)
    npad = g * BN
    f = jnp.pad(train_features, ((0, npad - n), (0, 0)))
    labs = jnp.pad(train_labels, (0, npad - n)).reshape(g, 1, BN)

    body = functools.partial(_body, n, g)
    return pl.pallas_call(
        body,
        grid=(g,),
        in_specs=[
            pl.BlockSpec((Q, D), lambda i: (0, 0)),
            pl.BlockSpec((BN, D), lambda i: (i, 0)),
            pl.BlockSpec((1, 1, BN), lambda i: (i, 0, 0)),
        ],
        out_specs=pl.BlockSpec((Q, C), lambda i: (0, 0)),
        out_shape=jax.ShapeDtypeStruct((Q, C), jnp.float32),
        scratch_shapes=[
            pltpu.VMEM((Q, 16), jnp.float32),
            pltpu.VMEM((Q, 16), jnp.float32),
        ],
        compiler_params=pltpu.CompilerParams(
            dimension_semantics=("arbitrary",),
        ),
    )(x, f, labs)
